# baseline (speedup 1.0000x reference)
import functools

import jax
import jax.numpy as jnp
from jax import lax
from jax.experimental import pallas as pl
from jax.experimental.pallas import tpu as pltpu
from jax.experimental.pallas import tpu_sc as plsc

N_NODES = 100000
N_EDGES = 3200000

NW = 32
LANES = 16
NP = 100352
NPW = NP // NW
NPS = NP // 16

ROWS_PER_W = 784
EP = NW * ROWS_PER_W * 128
CHUNK_ROWS = 16
N_ITERS = ROWS_PER_W // CHUNK_ROWS

_MESH = plsc.VectorSubcoreMesh(core_axis_name="c", subcore_axis_name="s")
_CPARAMS = pltpu.CompilerParams(needs_layout_passes=False)


def _worker_id():
    return lax.axis_index("s") * 2 + lax.axis_index("c")


@functools.partial(
    pl.kernel,
    mesh=_MESH,
    compiler_params=_CPARAMS,
    out_type=[
        jax.ShapeDtypeStruct((2 * NP,), jnp.float32),
        jax.ShapeDtypeStruct((2 * NP,), jnp.float32),
    ],
    scratch_types=[
        pltpu.VMEM((NP,), jnp.float32),
        pltpu.VMEM((3, LANES), jnp.float32),
        [pltpu.VMEM((CHUNK_ROWS, 128), jnp.int32) for _ in range(2)],
        [pltpu.VMEM((CHUNK_ROWS, 128), jnp.int32) for _ in range(2)],
        [pltpu.VMEM((CHUNK_ROWS, 128), jnp.float32) for _ in range(2)],
        [pltpu.VMEM((CHUNK_ROWS, 128), jnp.float32) for _ in range(2)],
        pltpu.VMEM_SHARED((NP,), jnp.float32),
        pltpu.VMEM_SHARED((NP,), jnp.float32),
        [pltpu.SemaphoreType.DMA for _ in range(2)],
        [pltpu.SemaphoreType.DMA for _ in range(2)],
    ],
)
def _edge_pass(val_hbm, srcm, dstm, par_hbm, zeros_hbm, den_out, num_out,
               val_v, par_v, src_v, dst_v, ex_v, exv_v, den_sp, num_sp,
               scat_sem, idx_sem):
    c = lax.axis_index("c")
    s = lax.axis_index("s")
    wid = s * 2 + c

    pltpu.sync_copy(par_hbm, par_v)
    pltpu.sync_copy(val_hbm, val_v)

    zoff = s * NPS
    pltpu.sync_copy(zeros_hbm.at[pl.ds(zoff, NPS)], den_sp.at[pl.ds(zoff, NPS)])
    pltpu.sync_copy(zeros_hbm.at[pl.ds(zoff, NPS)], num_sp.at[pl.ds(zoff, NPS)])
    plsc.subcore_barrier()

    csv = par_v[0, :]
    cdv = par_v[1, :]
    mv = par_v[2, :]

    row0 = wid * ROWS_PER_W

    def start_idx_load(i, b):
        rb = row0 + i * CHUNK_ROWS
        pltpu.async_copy(srcm.at[pl.ds(rb, CHUNK_ROWS)], src_v[b], idx_sem[b])
        pltpu.async_copy(dstm.at[pl.ds(rb, CHUNK_ROWS)], dst_v[b], idx_sem[b])

    def wait_idx_load(b):
        pltpu.make_async_copy(srcm.at[pl.ds(0, CHUNK_ROWS)], src_v[b],
                              idx_sem[b]).wait()
        pltpu.make_async_copy(dstm.at[pl.ds(0, CHUNK_ROWS)], dst_v[b],
                              idx_sem[b]).wait()

    def compute_chunk(b):
        for r in range(CHUNK_ROWS):
            for q in range(8):
                sl = pl.ds(q * LANES, LANES)
                si = src_v[b][r, sl]
                di = dst_v[b][r, sl]
                vs = plsc.load_gather(val_v, [si])
                vd = plsc.load_gather(val_v, [di])
                t = vs * csv + vd * cdv
                a = jnp.maximum(t, 0.2 * t)
                e = jnp.exp(a - mv)
                ex_v[b][r, sl] = e
                exv_v[b][r, sl] = e * vs

    def fire_scatters(b):
        for r in range(CHUNK_ROWS):
            pltpu.async_copy(
                ex_v[b].at[r], den_sp.at[dst_v[b].at[r]], scat_sem[b],
                add=True)
            pltpu.async_copy(
                exv_v[b].at[r], num_sp.at[dst_v[b].at[r]], scat_sem[b],
                add=True)

    def drain_scatters(b):
        for r in range(CHUNK_ROWS):
            pltpu.make_async_copy(
                ex_v[b].at[r], den_sp.at[dst_v[b].at[r]], scat_sem[b]).wait()
            pltpu.make_async_copy(
                exv_v[b].at[r], num_sp.at[dst_v[b].at[r]], scat_sem[b]).wait()

    pltpu.sync_copy(srcm.at[pl.ds(row0, CHUNK_ROWS)], src_v[0])
    pltpu.sync_copy(dstm.at[pl.ds(row0, CHUNK_ROWS)], dst_v[0])

    def pair_body(p, carry):
        compute_chunk(0)

        @pl.when(p > 0)
        def _():
            drain_scatters(1)

        start_idx_load(2 * p + 1, 1)
        fire_scatters(0)
        wait_idx_load(1)
        compute_chunk(1)
        drain_scatters(0)
        start_idx_load(2 * p + 2, 0)
        fire_scatters(1)
        wait_idx_load(0)
        return carry

    lax.fori_loop(0, (N_ITERS - 1) // 2, pair_body, 0)
    compute_chunk(0)
    drain_scatters(1)
    fire_scatters(0)
    drain_scatters(0)
    plsc.subcore_barrier()

    pltpu.sync_copy(den_sp.at[pl.ds(zoff, NPS)],
                    den_out.at[pl.ds(c * NP + zoff, NPS)])
    pltpu.sync_copy(num_sp.at[pl.ds(zoff, NPS)],
                    num_out.at[pl.ds(c * NP + zoff, NPS)])


@functools.partial(
    pl.kernel,
    mesh=_MESH,
    compiler_params=_CPARAMS,
    out_type=[jax.ShapeDtypeStruct((NP,), jnp.float32)],
    scratch_types=[
        pltpu.VMEM((NPW,), jnp.float32),
        pltpu.VMEM((NPW,), jnp.float32),
        pltpu.VMEM((NPW,), jnp.float32),
        pltpu.VMEM((NPW,), jnp.float32),
        pltpu.VMEM((NPW,), jnp.float32),
        pltpu.VMEM((NPW,), jnp.float32),
        pltpu.VMEM((2, LANES), jnp.float32),
        pltpu.VMEM((LANES, LANES), jnp.float32),
        pltpu.VMEM((LANES, LANES), jnp.float32),
        pltpu.VMEM((LANES, LANES), jnp.float32),
    ],
)
def _node_pass1(den_hbm, num_hbm, x_hbm, par_hbm, w1_hbm, b1_hbm, w2_hbm,
                g_out, d0_v, d1_v, n0_v, n1_v, x_v, g_v, par_v,
                w1_v, b1_v, w2_v):
    wid = _worker_id()
    base = wid * NPW

    pltpu.sync_copy(par_hbm, par_v)
    pltpu.sync_copy(w1_hbm, w1_v)
    pltpu.sync_copy(b1_hbm, b1_v)
    pltpu.sync_copy(w2_hbm, w2_v)
    pltpu.sync_copy(den_hbm.at[pl.ds(base, NPW)], d0_v)
    pltpu.sync_copy(den_hbm.at[pl.ds(NP + base, NPW)], d1_v)
    pltpu.sync_copy(num_hbm.at[pl.ds(base, NPW)], n0_v)
    pltpu.sync_copy(num_hbm.at[pl.ds(NP + base, NPW)], n1_v)
    pltpu.sync_copy(x_hbm.at[pl.ds(base, NPW)], x_v)

    csdv = par_v[0, :]
    mv = par_v[1, :]

    def body(j, carry):
        sl = pl.ds(j * LANES, LANES)
        xv = x_v[sl]
        den = d0_v[sl] + d1_v[sl]
        num = n0_v[sl] + n1_v[sl]
        t = csdv * xv
        a = jnp.maximum(t, 0.2 * t)
        e = jnp.exp(a - mv)
        den = den + e
        num = num + e * xv
        sres = num / (den + 1e-16)
        g = jnp.zeros((LANES,), jnp.float32)
        for k in range(LANES):
            g = g + w2_v[k, :] * jnp.maximum(
                sres * w1_v[k, :] + b1_v[k, :], 0.0)
        g_v[sl] = g
        return carry

    lax.fori_loop(0, NPW // LANES, body, 0)
    pltpu.sync_copy(g_v, g_out.at[pl.ds(base, NPW)])


@functools.partial(
    pl.kernel,
    mesh=_MESH,
    compiler_params=_CPARAMS,
    out_type=[jax.ShapeDtypeStruct((NP,), jnp.float32)],
    scratch_types=[
        pltpu.VMEM((NPW,), jnp.float32),
        pltpu.VMEM((NPW,), jnp.float32),
        pltpu.VMEM((NPW,), jnp.float32),
        pltpu.VMEM((NPW,), jnp.float32),
        pltpu.VMEM((NPW,), jnp.float32),
        pltpu.VMEM((NPW,), jnp.float32),
        pltpu.VMEM((3, LANES), jnp.float32),
    ],
)
def _node_pass2(den_hbm, num_hbm, g_hbm, par_hbm, out_hbm,
                d0_v, d1_v, n0_v, n1_v, g_v, o_v, par_v):
    wid = _worker_id()
    base = wid * NPW

    pltpu.sync_copy(par_hbm, par_v)
    pltpu.sync_copy(den_hbm.at[pl.ds(base, NPW)], d0_v)
    pltpu.sync_copy(den_hbm.at[pl.ds(NP + base, NPW)], d1_v)
    pltpu.sync_copy(num_hbm.at[pl.ds(base, NPW)], n0_v)
    pltpu.sync_copy(num_hbm.at[pl.ds(NP + base, NPW)], n1_v)
    pltpu.sync_copy(g_hbm.at[pl.ds(base, NPW)], g_v)

    csdv = par_v[0, :]
    mv = par_v[1, :]
    bv = par_v[2, :]

    def body(j, carry):
        sl = pl.ds(j * LANES, LANES)
        gv = g_v[sl]
        den = d0_v[sl] + d1_v[sl]
        num = n0_v[sl] + n1_v[sl]
        t = csdv * gv
        a = jnp.maximum(t, 0.2 * t)
        e = jnp.exp(a - mv)
        den = den + e
        num = num + e * gv
        o_v[sl] = num / (den + 1e-16) + bv
        return carry

    lax.fori_loop(0, NPW // LANES, body, 0)
    pltpu.sync_copy(o_v, out_hbm.at[pl.ds(base, NPW)])


def _splat(v):
    return jnp.full((LANES,), v, jnp.float32)


def kernel(x, edge_index, W1, a_src1, a_dst1, b1, W2, a_src2, a_dst2, b2):
    N = x.shape[0]
    E = edge_index.shape[1]

    xv = jnp.pad(x[:, 0].astype(jnp.float32), (0, NP - N))
    ei = edge_index.astype(jnp.int32)
    pad = EP - E
    srcm = jnp.concatenate(
        [ei[0], jnp.full((pad,), N, jnp.int32)]).reshape(EP // 128, 128)
    dstm = jnp.concatenate(
        [ei[1], jnp.full((pad,), N, jnp.int32)]).reshape(EP // 128, 128)
    zeros_np = jnp.zeros((NP,), jnp.float32)

    cs1 = jnp.dot(W1[0], a_src1)
    cd1 = jnp.dot(W1[0], a_dst1)
    m1 = jnp.maximum(0.0, jnp.max(cs1 * xv) + jnp.max(cd1 * xv))
    par1 = jnp.stack([_splat(cs1), _splat(cd1), _splat(m1)])
    par1n = jnp.stack([_splat(cs1 + cd1), _splat(m1)])

    xmax = jnp.max(jnp.abs(xv))
    gb = jnp.sum(jnp.abs(W2[:, 0]) * (xmax * jnp.abs(W1[0]) + jnp.abs(b1)))
    cs2 = a_src2[0]
    cd2 = a_dst2[0]
    m2 = jnp.maximum(0.0, (jnp.abs(cs2) + jnp.abs(cd2)) * gb)
    par2 = jnp.stack([_splat(cs2), _splat(cd2), _splat(m2)])
    par2n = jnp.stack([_splat(cs2 + cd2), _splat(m2), _splat(b2[0])])

    w1s = jnp.broadcast_to(W1[0][:, None], (LANES, LANES)).astype(jnp.float32)
    b1s = jnp.broadcast_to(b1[:, None], (LANES, LANES)).astype(jnp.float32)
    w2s = jnp.broadcast_to(W2[:, 0][:, None], (LANES, LANES)).astype(jnp.float32)

    den1, num1 = _edge_pass(xv, srcm, dstm, par1, zeros_np)
    (g,) = _node_pass1(den1, num1, xv, par1n, w1s, b1s, w2s)
    den2, num2 = _edge_pass(g, srcm, dstm, par2, zeros_np)
    (o,) = _node_pass2(den2, num2, g, par2n)

    out = o[:N].reshape(N, 1)
    return jax.nn.log_softmax(out, axis=1)


# speedup vs baseline: 2.0833x; 2.0833x over previous
import functools

import jax
import jax.numpy as jnp
from jax import lax
from jax.experimental import pallas as pl
from jax.experimental.pallas import tpu as pltpu
from jax.experimental.pallas import tpu_sc as plsc

N_NODES = 100000
N_EDGES = 3200000

NW = 32
LANES = 16
NP = 100352
NPW = NP // NW
NPS = NP // 16

ROWS_PER_W = 784
EP = NW * ROWS_PER_W * 128
CHUNK_ROWS = 16
N_ITERS = ROWS_PER_W // CHUNK_ROWS

_MESH = plsc.VectorSubcoreMesh(core_axis_name="c", subcore_axis_name="s")
_CPARAMS = pltpu.CompilerParams(needs_layout_passes=False)


def _worker_id():
    return lax.axis_index("s") * 2 + lax.axis_index("c")


@functools.partial(
    pl.kernel,
    mesh=_MESH,
    compiler_params=_CPARAMS,
    out_type=[
        jax.ShapeDtypeStruct((2 * NP,), jnp.float32),
        jax.ShapeDtypeStruct((2 * NP,), jnp.float32),
    ],
    scratch_types=[
        pltpu.VMEM((NP,), jnp.float32),
        pltpu.VMEM((3, LANES), jnp.float32),
        [pltpu.VMEM((CHUNK_ROWS, 128), jnp.int32) for _ in range(2)],
        [pltpu.VMEM((CHUNK_ROWS, 128), jnp.int32) for _ in range(2)],
        [pltpu.VMEM((CHUNK_ROWS, 128), jnp.float32) for _ in range(2)],
        [pltpu.VMEM((CHUNK_ROWS, 128), jnp.float32) for _ in range(2)],
        pltpu.VMEM_SHARED((NP,), jnp.float32),
        pltpu.VMEM_SHARED((NP,), jnp.float32),
        [pltpu.SemaphoreType.DMA for _ in range(2)],
        [pltpu.SemaphoreType.DMA for _ in range(2)],
    ],
)
def _edge_pass(val_hbm, srcm, dstm, par_hbm, zeros_hbm, den_out, num_out,
               val_v, par_v, src_v, dst_v, ex_v, exv_v, den_sp, num_sp,
               scat_sem, idx_sem):
    c = lax.axis_index("c")
    s = lax.axis_index("s")
    wid = s * 2 + c

    pltpu.sync_copy(par_hbm, par_v)
    pltpu.sync_copy(val_hbm, val_v)

    zoff = s * NPS
    pltpu.sync_copy(zeros_hbm.at[pl.ds(zoff, NPS)], den_sp.at[pl.ds(zoff, NPS)])
    pltpu.sync_copy(zeros_hbm.at[pl.ds(zoff, NPS)], num_sp.at[pl.ds(zoff, NPS)])
    plsc.subcore_barrier()

    csv = par_v[0, :]
    cdv = par_v[1, :]
    mv = par_v[2, :]

    row0 = wid * ROWS_PER_W

    def start_idx_load(i, b):
        rb = row0 + i * CHUNK_ROWS
        pltpu.async_copy(srcm.at[pl.ds(rb, CHUNK_ROWS)], src_v[b], idx_sem[b])
        pltpu.async_copy(dstm.at[pl.ds(rb, CHUNK_ROWS)], dst_v[b], idx_sem[b])

    def wait_idx_load(b):
        pltpu.make_async_copy(srcm.at[pl.ds(0, CHUNK_ROWS)], src_v[b],
                              idx_sem[b]).wait()
        pltpu.make_async_copy(dstm.at[pl.ds(0, CHUNK_ROWS)], dst_v[b],
                              idx_sem[b]).wait()

    def compute_chunk(b):
        @plsc.parallel_loop(0, CHUNK_ROWS * 8, unroll=8)
        def _(gi):
            r = lax.shift_right_logical(gi, 3)
            q = lax.bitwise_and(gi, 7)
            sl = pl.ds(q * LANES, LANES)
            si = src_v[b][r, sl]
            di = dst_v[b][r, sl]
            vs = plsc.load_gather(val_v, [si])
            vd = plsc.load_gather(val_v, [di])
            t = vs * csv + vd * cdv
            a = jnp.maximum(t, 0.2 * t)
            e = jnp.exp(a - mv)
            ex_v[b][r, sl] = e
            exv_v[b][r, sl] = e * vs

    def fire_scatters(b):
        for r in range(CHUNK_ROWS):
            pltpu.async_copy(
                ex_v[b].at[r], den_sp.at[dst_v[b].at[r]], scat_sem[b],
                add=True)
            pltpu.async_copy(
                exv_v[b].at[r], num_sp.at[dst_v[b].at[r]], scat_sem[b],
                add=True)

    def drain_scatters(b):
        for r in range(CHUNK_ROWS):
            pltpu.make_async_copy(
                ex_v[b].at[r], den_sp.at[dst_v[b].at[r]], scat_sem[b]).wait()
            pltpu.make_async_copy(
                exv_v[b].at[r], num_sp.at[dst_v[b].at[r]], scat_sem[b]).wait()

    pltpu.sync_copy(srcm.at[pl.ds(row0, CHUNK_ROWS)], src_v[0])
    pltpu.sync_copy(dstm.at[pl.ds(row0, CHUNK_ROWS)], dst_v[0])

    def pair_body(p, carry):
        compute_chunk(0)

        @pl.when(p > 0)
        def _():
            drain_scatters(1)

        start_idx_load(2 * p + 1, 1)
        fire_scatters(0)
        wait_idx_load(1)
        compute_chunk(1)
        drain_scatters(0)
        start_idx_load(2 * p + 2, 0)
        fire_scatters(1)
        wait_idx_load(0)
        return carry

    lax.fori_loop(0, (N_ITERS - 1) // 2, pair_body, 0)
    compute_chunk(0)
    drain_scatters(1)
    fire_scatters(0)
    drain_scatters(0)
    plsc.subcore_barrier()

    pltpu.sync_copy(den_sp.at[pl.ds(zoff, NPS)],
                    den_out.at[pl.ds(c * NP + zoff, NPS)])
    pltpu.sync_copy(num_sp.at[pl.ds(zoff, NPS)],
                    num_out.at[pl.ds(c * NP + zoff, NPS)])


@functools.partial(
    pl.kernel,
    mesh=_MESH,
    compiler_params=_CPARAMS,
    out_type=[jax.ShapeDtypeStruct((NP,), jnp.float32)],
    scratch_types=[
        pltpu.VMEM((NPW,), jnp.float32),
        pltpu.VMEM((NPW,), jnp.float32),
        pltpu.VMEM((NPW,), jnp.float32),
        pltpu.VMEM((NPW,), jnp.float32),
        pltpu.VMEM((NPW,), jnp.float32),
        pltpu.VMEM((NPW,), jnp.float32),
        pltpu.VMEM((2, LANES), jnp.float32),
        pltpu.VMEM((LANES, LANES), jnp.float32),
        pltpu.VMEM((LANES, LANES), jnp.float32),
        pltpu.VMEM((LANES, LANES), jnp.float32),
    ],
)
def _node_pass1(den_hbm, num_hbm, x_hbm, par_hbm, w1_hbm, b1_hbm, w2_hbm,
                g_out, d0_v, d1_v, n0_v, n1_v, x_v, g_v, par_v,
                w1_v, b1_v, w2_v):
    wid = _worker_id()
    base = wid * NPW

    pltpu.sync_copy(par_hbm, par_v)
    pltpu.sync_copy(w1_hbm, w1_v)
    pltpu.sync_copy(b1_hbm, b1_v)
    pltpu.sync_copy(w2_hbm, w2_v)
    pltpu.sync_copy(den_hbm.at[pl.ds(base, NPW)], d0_v)
    pltpu.sync_copy(den_hbm.at[pl.ds(NP + base, NPW)], d1_v)
    pltpu.sync_copy(num_hbm.at[pl.ds(base, NPW)], n0_v)
    pltpu.sync_copy(num_hbm.at[pl.ds(NP + base, NPW)], n1_v)
    pltpu.sync_copy(x_hbm.at[pl.ds(base, NPW)], x_v)

    csdv = par_v[0, :]
    mv = par_v[1, :]

    def body(j, carry):
        sl = pl.ds(j * LANES, LANES)
        xv = x_v[sl]
        den = d0_v[sl] + d1_v[sl]
        num = n0_v[sl] + n1_v[sl]
        t = csdv * xv
        a = jnp.maximum(t, 0.2 * t)
        e = jnp.exp(a - mv)
        den = den + e
        num = num + e * xv
        sres = num / (den + 1e-16)
        g = jnp.zeros((LANES,), jnp.float32)
        for k in range(LANES):
            g = g + w2_v[k, :] * jnp.maximum(
                sres * w1_v[k, :] + b1_v[k, :], 0.0)
        g_v[sl] = g
        return carry

    lax.fori_loop(0, NPW // LANES, body, 0)
    pltpu.sync_copy(g_v, g_out.at[pl.ds(base, NPW)])


@functools.partial(
    pl.kernel,
    mesh=_MESH,
    compiler_params=_CPARAMS,
    out_type=[jax.ShapeDtypeStruct((NP,), jnp.float32)],
    scratch_types=[
        pltpu.VMEM((NPW,), jnp.float32),
        pltpu.VMEM((NPW,), jnp.float32),
        pltpu.VMEM((NPW,), jnp.float32),
        pltpu.VMEM((NPW,), jnp.float32),
        pltpu.VMEM((NPW,), jnp.float32),
        pltpu.VMEM((NPW,), jnp.float32),
        pltpu.VMEM((3, LANES), jnp.float32),
    ],
)
def _node_pass2(den_hbm, num_hbm, g_hbm, par_hbm, out_hbm,
                d0_v, d1_v, n0_v, n1_v, g_v, o_v, par_v):
    wid = _worker_id()
    base = wid * NPW

    pltpu.sync_copy(par_hbm, par_v)
    pltpu.sync_copy(den_hbm.at[pl.ds(base, NPW)], d0_v)
    pltpu.sync_copy(den_hbm.at[pl.ds(NP + base, NPW)], d1_v)
    pltpu.sync_copy(num_hbm.at[pl.ds(base, NPW)], n0_v)
    pltpu.sync_copy(num_hbm.at[pl.ds(NP + base, NPW)], n1_v)
    pltpu.sync_copy(g_hbm.at[pl.ds(base, NPW)], g_v)

    csdv = par_v[0, :]
    mv = par_v[1, :]
    bv = par_v[2, :]

    def body(j, carry):
        sl = pl.ds(j * LANES, LANES)
        gv = g_v[sl]
        den = d0_v[sl] + d1_v[sl]
        num = n0_v[sl] + n1_v[sl]
        t = csdv * gv
        a = jnp.maximum(t, 0.2 * t)
        e = jnp.exp(a - mv)
        den = den + e
        num = num + e * gv
        o_v[sl] = num / (den + 1e-16) + bv
        return carry

    lax.fori_loop(0, NPW // LANES, body, 0)
    pltpu.sync_copy(o_v, out_hbm.at[pl.ds(base, NPW)])


def _splat(v):
    return jnp.full((LANES,), v, jnp.float32)


def kernel(x, edge_index, W1, a_src1, a_dst1, b1, W2, a_src2, a_dst2, b2):
    N = x.shape[0]
    E = edge_index.shape[1]

    xv = jnp.pad(x[:, 0].astype(jnp.float32), (0, NP - N))
    ei = edge_index.astype(jnp.int32)
    pad = EP - E
    srcm = jnp.concatenate(
        [ei[0], jnp.full((pad,), N, jnp.int32)]).reshape(EP // 128, 128)
    dstm = jnp.concatenate(
        [ei[1], jnp.full((pad,), N, jnp.int32)]).reshape(EP // 128, 128)
    zeros_np = jnp.zeros((NP,), jnp.float32)

    cs1 = jnp.dot(W1[0], a_src1)
    cd1 = jnp.dot(W1[0], a_dst1)
    m1 = jnp.maximum(0.0, jnp.max(cs1 * xv) + jnp.max(cd1 * xv))
    par1 = jnp.stack([_splat(cs1), _splat(cd1), _splat(m1)])
    par1n = jnp.stack([_splat(cs1 + cd1), _splat(m1)])

    xmax = jnp.max(jnp.abs(xv))
    gb = jnp.sum(jnp.abs(W2[:, 0]) * (xmax * jnp.abs(W1[0]) + jnp.abs(b1)))
    cs2 = a_src2[0]
    cd2 = a_dst2[0]
    m2 = jnp.maximum(0.0, (jnp.abs(cs2) + jnp.abs(cd2)) * gb)
    par2 = jnp.stack([_splat(cs2), _splat(cd2), _splat(m2)])
    par2n = jnp.stack([_splat(cs2 + cd2), _splat(m2), _splat(b2[0])])

    w1s = jnp.broadcast_to(W1[0][:, None], (LANES, LANES)).astype(jnp.float32)
    b1s = jnp.broadcast_to(b1[:, None], (LANES, LANES)).astype(jnp.float32)
    w2s = jnp.broadcast_to(W2[:, 0][:, None], (LANES, LANES)).astype(jnp.float32)

    den1, num1 = _edge_pass(xv, srcm, dstm, par1, zeros_np)
    (g,) = _node_pass1(den1, num1, xv, par1n, w1s, b1s, w2s)
    den2, num2 = _edge_pass(g, srcm, dstm, par2, zeros_np)
    (o,) = _node_pass2(den2, num2, g, par2n)

    out = o[:N].reshape(N, 1)
    return jax.nn.log_softmax(out, axis=1)
